# baseline (speedup 1.0000x reference)
import functools

import jax
import jax.numpy as jnp
from jax import lax
from jax.experimental import pallas as pl
from jax.experimental.pallas import tpu as pltpu
from jax.experimental.pallas import tpu_sc as plsc

_NUM_CORES = 2
_NUM_SUBCORES = 16
_LANES = 16
_NUM_WORKERS = _NUM_CORES * _NUM_SUBCORES

_LATENT = 64
_CHUNK = 128


def _make_kernel(batch: int, n_users: int, n_products: int):
    b_per_w = batch // _NUM_WORKERS
    n_chunks = b_per_w // _CHUNK
    vpc = _CHUNK // _LANES
    flat_n = _LATENT * _CHUNK

    mesh = plsc.VectorSubcoreMesh(
        core_axis_name="c",
        subcore_axis_name="s",
        num_cores=_NUM_CORES,
        num_subcores=_NUM_SUBCORES,
    )

    @functools.partial(
        pl.kernel,
        mesh=mesh,
        out_type=jax.ShapeDtypeStruct((batch,), jnp.float32),
        scratch_types=[
            pltpu.VMEM((n_chunks, _CHUNK), jnp.int32),
            pltpu.VMEM((n_chunks, _CHUNK), jnp.int32),
            pltpu.VMEM((flat_n,), jnp.int32),
            pltpu.VMEM((flat_n,), jnp.int32),
            pltpu.VMEM((flat_n,), jnp.float32),
            pltpu.VMEM((flat_n,), jnp.float32),
            pltpu.VMEM((b_per_w,), jnp.float32),
            pltpu.SemaphoreType.DMA,
        ],
        compiler_params=pltpu.CompilerParams(use_tc_tiling_on_sc=False),
    )
    def k(uidx_hbm, pidx_hbm, ut_hbm, pt_hbm, out_hbm,
          uidx_v, pidx_v, uflat, pflat, uvals, pvals, outv, sem):
        wid = lax.axis_index("s") * _NUM_CORES + lax.axis_index("c")
        base = wid * b_per_w

        for j in range(n_chunks):
            pltpu.sync_copy(
                uidx_hbm.at[pl.ds(base + j * _CHUNK, _CHUNK)], uidx_v.at[j])
            pltpu.sync_copy(
                pidx_hbm.at[pl.ds(base + j * _CHUNK, _CHUNK)], pidx_v.at[j])

        def chunk_body(j, _):
            def gen_body(d, _):
                for v in range(vpc):
                    ui = uidx_v[j, pl.ds(v * _LANES, _LANES)]
                    pi = pidx_v[j, pl.ds(v * _LANES, _LANES)]
                    off = d * _CHUNK + v * _LANES
                    uflat[pl.ds(off, _LANES)] = ui + d * n_users
                    pflat[pl.ds(off, _LANES)] = pi + d * n_products
                return 0

            lax.fori_loop(0, _LATENT, gen_body, 0)

            cu = pltpu.async_copy(ut_hbm.at[uflat], uvals, sem)
            cp = pltpu.async_copy(pt_hbm.at[pflat], pvals, sem)
            cu.wait()
            cp.wait()

            def dot_body(d, accs):
                new = []
                for v in range(vpc):
                    off = d * _CHUNK + v * _LANES
                    u = uvals[pl.ds(off, _LANES)]
                    p = pvals[pl.ds(off, _LANES)]
                    new.append(accs[v] + u * p)
                return tuple(new)

            accs = lax.fori_loop(
                0, _LATENT, dot_body,
                tuple(jnp.zeros((_LANES,), jnp.float32) for _ in range(vpc)))
            for v in range(vpc):
                res = 1.0 / (1.0 + jnp.exp(-accs[v]))
                outv[pl.ds(j * _CHUNK + v * _LANES, _LANES)] = res
            return 0

        lax.fori_loop(0, n_chunks, chunk_body, 0)

        pltpu.sync_copy(outv, out_hbm.at[pl.ds(base, b_per_w)])

    return k


@jax.jit
def kernel(inputs, user_table, product_table):
    batch = inputs.shape[0]
    uidx = inputs[:, 0].astype(jnp.int32)
    pidx = inputs[:, 1].astype(jnp.int32)
    ut_flat = user_table.T.reshape(-1)
    pt_flat = product_table.T.reshape(-1)
    k = _make_kernel(batch, user_table.shape[0], product_table.shape[0])
    return k(uidx, pidx, ut_flat, pt_flat)


# speedup vs baseline: 9.1076x; 9.1076x over previous
import functools

import jax
import jax.numpy as jnp
from jax import lax
from jax.experimental import pallas as pl
from jax.experimental.pallas import tpu as pltpu
from jax.experimental.pallas import tpu_sc as plsc

_NUM_CORES = 2
_NUM_SUBCORES = 16
_LANES = 16
_NUM_WORKERS = _NUM_CORES * _NUM_SUBCORES

_LATENT = 64
_PAIR = 2 * _LATENT
_CHUNK = 128


def _perm(x, idx):
    return lax.gather(
        x, idx[:, None],
        dimension_numbers=lax.GatherDimensionNumbers(
            offset_dims=(), collapsed_slice_dims=(0,), start_index_map=(0,)),
        slice_sizes=(1,),
        mode=lax.GatherScatterMode.PROMISE_IN_BOUNDS)


def _make_kernel(batch: int):
    b_per_w = batch // _NUM_WORKERS
    n_chunks = b_per_w // _CHUNK
    n_groups = b_per_w // _LANES

    mesh = plsc.VectorSubcoreMesh(
        core_axis_name="c",
        subcore_axis_name="s",
        num_cores=_NUM_CORES,
        num_subcores=_NUM_SUBCORES,
    )

    @functools.partial(
        pl.kernel,
        mesh=mesh,
        out_type=jax.ShapeDtypeStruct((batch,), jnp.float32),
        scratch_types=[
            pltpu.VMEM((b_per_w,), jnp.int32),
            pltpu.VMEM((b_per_w,), jnp.int32),
            pltpu.VMEM((b_per_w,), jnp.int32),
            pltpu.VMEM((b_per_w,), jnp.int32),
            pltpu.VMEM((2, _CHUNK, _PAIR), jnp.float32),
            pltpu.VMEM((2, _CHUNK, _PAIR), jnp.float32),
            pltpu.VMEM((b_per_w,), jnp.float32),
            pltpu.SemaphoreType.DMA,
        ],
    )
    def k(urow_hbm, prow_hbm, uoff_hbm, poff_hbm, ut_hbm, pt_hbm, out_hbm,
          urow_v, prow_v, uoff_v, poff_v, urows, prows, outv, sem):
        wid = lax.axis_index("s") * _NUM_CORES + lax.axis_index("c")
        base = wid * b_per_w

        pltpu.sync_copy(urow_hbm.at[pl.ds(base, b_per_w)], urow_v)
        pltpu.sync_copy(prow_hbm.at[pl.ds(base, b_per_w)], prow_v)
        pltpu.sync_copy(uoff_hbm.at[pl.ds(base, b_per_w)], uoff_v)
        pltpu.sync_copy(poff_hbm.at[pl.ds(base, b_per_w)], poff_v)

        lane = lax.iota(jnp.int32, _LANES)
        groups_per_chunk = _CHUNK // _LANES

        def fire(c):
            slot = c % 2
            pltpu.async_copy(
                ut_hbm.at[urow_v.at[pl.ds(c * _CHUNK, _CHUNK)]],
                urows.at[slot], sem)
            pltpu.async_copy(
                pt_hbm.at[prow_v.at[pl.ds(c * _CHUNK, _CHUNK)]],
                prows.at[slot], sem)

        def drain(c):
            slot = c % 2
            pltpu.make_async_copy(
                ut_hbm.at[pl.ds(0, _CHUNK)], urows.at[slot], sem).wait()
            pltpu.make_async_copy(
                pt_hbm.at[pl.ds(0, _CHUNK)], prows.at[slot], sem).wait()

        def compute(c):
            slot = c % 2

            def group_body(g, _):
                b0 = c * _CHUNK + g * _LANES
                uo_vec = uoff_v[pl.ds(b0, _LANES)]
                po_vec = poff_v[pl.ds(b0, _LANES)]
                res = jnp.zeros((_LANES,), jnp.float32)
                for j in range(_LANES):
                    r = g * _LANES + j
                    uo = uo_vec[j]
                    po = po_vec[j]
                    acc = jnp.zeros((_LANES,), jnp.float32)
                    for v in range(_LATENT // _LANES):
                        u = urows[slot, r, pl.ds(uo + v * _LANES, _LANES)]
                        p = prows[slot, r, pl.ds(po + v * _LANES, _LANES)]
                        acc = acc + u * p
                    for step in (8, 4, 2, 1):
                        acc = acc + _perm(acc, lane ^ step)
                    res = jnp.where(lane == j, acc, res)
                res = 1.0 / (1.0 + jnp.exp(-res))
                outv[pl.ds(b0, _LANES)] = res
                return 0

            lax.fori_loop(0, groups_per_chunk, group_body, 0)

        fire(0)

        def pipe_body(c, _):
            fire(c)
            drain(c - 1)
            compute(c - 1)
            return 0

        lax.fori_loop(1, n_chunks, pipe_body, 0)
        drain(n_chunks - 1)
        compute(n_chunks - 1)

        pltpu.sync_copy(outv, out_hbm.at[pl.ds(base, b_per_w)])

    return k


@jax.jit
def kernel(inputs, user_table, product_table):
    batch = inputs.shape[0]
    uidx = inputs[:, 0].astype(jnp.int32)
    pidx = inputs[:, 1].astype(jnp.int32)
    urow = uidx >> 1
    prow = pidx >> 1
    uoff = (uidx & 1) * _LATENT
    poff = (pidx & 1) * _LATENT
    ut2 = user_table.reshape(user_table.shape[0] // 2, _PAIR)
    pt2 = product_table.reshape(product_table.shape[0] // 2, _PAIR)
    k = _make_kernel(batch)
    return k(urow, prow, uoff, poff, ut2, pt2)


# speedup vs baseline: 13.6125x; 1.4946x over previous
import functools

import jax
import jax.numpy as jnp
from jax import lax
from jax.experimental import pallas as pl
from jax.experimental.pallas import tpu as pltpu
from jax.experimental.pallas import tpu_sc as plsc

_NUM_CORES = 2
_NUM_SUBCORES = 16
_LANES = 16
_NUM_WORKERS = _NUM_CORES * _NUM_SUBCORES

_LATENT = 64
_TILE_ROWS = 8


def _perm(x, idx):
    return lax.gather(
        x, idx[:, None],
        dimension_numbers=lax.GatherDimensionNumbers(
            offset_dims=(), collapsed_slice_dims=(0,), start_index_map=(0,)),
        slice_sizes=(1,),
        mode=lax.GatherScatterMode.PROMISE_IN_BOUNDS)


def _make_kernel(batch: int):
    b_per_w = batch // _NUM_WORKERS
    n_groups = b_per_w // _LANES

    mesh = plsc.VectorSubcoreMesh(
        core_axis_name="c",
        subcore_axis_name="s",
        num_cores=_NUM_CORES,
        num_subcores=_NUM_SUBCORES,
    )

    @functools.partial(
        pl.kernel,
        mesh=mesh,
        out_type=jax.ShapeDtypeStruct((batch,), jnp.float32),
        scratch_types=[
            pltpu.VMEM((b_per_w,), jnp.int32),
            pltpu.VMEM((b_per_w,), jnp.int32),
            pltpu.VMEM((2, _LANES, _TILE_ROWS, _LATENT), jnp.float32),
            pltpu.VMEM((2, _LANES, _TILE_ROWS, _LATENT), jnp.float32),
            pltpu.VMEM((b_per_w,), jnp.float32),
            pltpu.SemaphoreType.DMA,
            pltpu.SemaphoreType.DMA,
        ],
    )
    def k(uidx_hbm, pidx_hbm, ut_hbm, pt_hbm, out_hbm,
          uidx_s, pidx_s, ublk, pblk, outv, usem, psem):
        wid = lax.axis_index("s") * _NUM_CORES + lax.axis_index("c")
        base = wid * b_per_w

        pltpu.sync_copy(uidx_hbm.at[pl.ds(base, b_per_w)], uidx_s)
        pltpu.sync_copy(pidx_hbm.at[pl.ds(base, b_per_w)], pidx_s)

        lane = lax.iota(jnp.int32, _LANES)

        def fire_group(g):
            slot = g % 2
            uvec = uidx_s[pl.ds(g * _LANES, _LANES)]
            pvec = pidx_s[pl.ds(g * _LANES, _LANES)]
            for j in range(_LANES):
                tu = (uvec[j] >> 3) * _TILE_ROWS
                tp = (pvec[j] >> 3) * _TILE_ROWS
                pltpu.async_copy(
                    ut_hbm.at[pl.ds(tu, _TILE_ROWS)], ublk.at[slot, j], usem)
                pltpu.async_copy(
                    pt_hbm.at[pl.ds(tp, _TILE_ROWS)], pblk.at[slot, j], psem)

        def drain_group(g):
            slot = g % 2
            for j in range(_LANES):
                pltpu.make_async_copy(
                    ut_hbm.at[pl.ds(0, _TILE_ROWS)],
                    ublk.at[slot, j], usem).wait()
                pltpu.make_async_copy(
                    pt_hbm.at[pl.ds(0, _TILE_ROWS)],
                    pblk.at[slot, j], psem).wait()

        def compute_group(g):
            slot = g % 2
            uvec = uidx_s[pl.ds(g * _LANES, _LANES)]
            pvec = pidx_s[pl.ds(g * _LANES, _LANES)]
            res = jnp.zeros((_LANES,), jnp.float32)
            for j in range(_LANES):
                ru = uvec[j] & (_TILE_ROWS - 1)
                rp = pvec[j] & (_TILE_ROWS - 1)
                acc = jnp.zeros((_LANES,), jnp.float32)
                for v in range(_LATENT // _LANES):
                    u = ublk[slot, j, ru, pl.ds(v * _LANES, _LANES)]
                    p = pblk[slot, j, rp, pl.ds(v * _LANES, _LANES)]
                    acc = acc + u * p
                for step in (8, 4, 2, 1):
                    acc = acc + _perm(acc, lane ^ step)
                res = jnp.where(lane == j, acc, res)
            res = 1.0 / (1.0 + jnp.exp(-res))
            outv[pl.ds(g * _LANES, _LANES)] = res

        fire_group(0)

        def pipe_body(g, _):
            fire_group(g)
            drain_group(g - 1)
            compute_group(g - 1)
            return 0

        lax.fori_loop(1, n_groups, pipe_body, 0)
        drain_group(n_groups - 1)
        compute_group(n_groups - 1)

        pltpu.sync_copy(outv, out_hbm.at[pl.ds(base, b_per_w)])

    return k


@jax.jit
def kernel(inputs, user_table, product_table):
    batch = inputs.shape[0]
    uidx = inputs[:, 0].astype(jnp.int32)
    pidx = inputs[:, 1].astype(jnp.int32)
    k = _make_kernel(batch)
    return k(uidx, pidx, user_table, product_table)
